# baseline (speedup 1.0000x reference)
import functools

import jax
import jax.numpy as jnp
from jax import lax
from jax.experimental import pallas as pl
from jax.experimental.pallas import tpu as pltpu
from jax.experimental.pallas import tpu_sc as plsc

N, DIM, K = 4096, 32, 8192
D2 = 2 * DIM
NT = 512
KC = 1024

_info = plsc.get_sparse_core_info()
NC = _info.num_cores
NS = _info.num_subcores
NW = NC * NS
BPW = N // NW


def _argmin_body(z_ref, w_ref, idx_ref):
    z = z_ref[...]
    x2 = jnp.sum(z * z, axis=1, keepdims=True)
    best_d = jnp.full((NT, 1), jnp.inf, dtype=jnp.float32)
    best_i = jnp.zeros((NT, 1), dtype=jnp.int32)
    for j in range(K // KC):
        wc = w_ref[pl.ds(j * KC, KC), :]
        w2 = jnp.sum(wc * wc, axis=1).reshape(1, KC)
        s = lax.dot_general(z, wc, (((1,), (1,)), ((), ())),
                            preferred_element_type=jnp.float32)
        d2 = (x2 + w2) - 2.0 * s
        d = jnp.sqrt(jnp.maximum(d2, 0.0))
        m = jnp.min(d, axis=1, keepdims=True)
        iota = lax.broadcasted_iota(jnp.int32, (NT, KC), 1) + j * KC
        cand = jnp.min(jnp.where(d == m, iota, K), axis=1, keepdims=True)
        upd = m < best_d
        best_d = jnp.where(upd, m, best_d)
        best_i = jnp.where(upd, cand, best_i)
    idx_ref[...] = best_i


def _argmin_call(z_flat, W):
    return pl.pallas_call(
        _argmin_body,
        grid=(N // NT,),
        in_specs=[
            pl.BlockSpec((NT, D2), lambda i: (i, 0)),
            pl.BlockSpec((K, D2), lambda i: (0, 0)),
        ],
        out_specs=pl.BlockSpec((NT, 1), lambda i: (i, 0)),
        out_shape=jax.ShapeDtypeStruct((N, 1), jnp.int32),
    )(z_flat, W)


def _gather_hist_body(w_hbm, idx_hbm, zeros_hbm, ones_hbm,
                      zq_out, cnt_out,
                      idx_v, rows_v, ones_v, cnt_sh, sem):
    c = lax.axis_index("c")
    s = lax.axis_index("s")
    wid = s * NC + c
    base = wid * BPW
    pltpu.sync_copy(idx_hbm.at[pl.ds(base, BPW)], idx_v)
    pltpu.async_copy(w_hbm.at[idx_v], rows_v, sem).wait()
    pltpu.sync_copy(rows_v, zq_out.at[pl.ds(base, BPW)])
    pltpu.sync_copy(ones_hbm, ones_v)

    @pl.when(s == 0)
    def _():
        pltpu.sync_copy(zeros_hbm, cnt_sh)

    plsc.subcore_barrier()
    pltpu.sync_copy(ones_v, cnt_sh.at[idx_v], add=True)
    plsc.subcore_barrier()

    @pl.when(s == 0)
    def _():
        pltpu.sync_copy(cnt_sh, cnt_out.at[c])


def _gather_hist_call(W, idx, zeros_k, ones_b):
    fn = functools.partial(
        pl.kernel,
        mesh=plsc.VectorSubcoreMesh(core_axis_name="c", subcore_axis_name="s"),
        out_type=[
            jax.ShapeDtypeStruct((N, D2), jnp.float32),
            jax.ShapeDtypeStruct((NC, K), jnp.float32),
        ],
        scratch_types=[
            pltpu.VMEM((BPW,), jnp.int32),
            pltpu.VMEM((BPW, D2), jnp.float32),
            pltpu.VMEM((BPW,), jnp.float32),
            pltpu.VMEM_SHARED((K,), jnp.float32),
            pltpu.SemaphoreType.DMA,
        ],
        compiler_params=pltpu.CompilerParams(use_tc_tiling_on_sc=False),
    )(_gather_hist_body)
    return fn(W, idx, zeros_k, ones_b)


def _loss_body(z_ref, zq_ref, cnt_ref, loss_ref, perp_ref):
    z = z_ref[...]
    q = zq_ref[...]
    dd = q - z
    ssq = jnp.sum(dd * dd)
    loss_ref[...] = jnp.full((1, 1), 1.25 / (N * D2), jnp.float32) * ssq
    p = jnp.sum(cnt_ref[...], axis=0, keepdims=True) * (1.0 / N)
    ent = jnp.sum(p * jnp.log(p + 1e-10))
    perp_ref[...] = jnp.full((1, 1), 1.0, jnp.float32) * jnp.exp(-ent)


def _loss_call(z_flat, zq, cnt):
    return pl.pallas_call(
        _loss_body,
        in_specs=[
            pl.BlockSpec((N, D2), lambda: (0, 0)),
            pl.BlockSpec((N, D2), lambda: (0, 0)),
            pl.BlockSpec((NC, K), lambda: (0, 0)),
        ],
        out_specs=[
            pl.BlockSpec((1, 1), lambda: (0, 0)),
            pl.BlockSpec((1, 1), lambda: (0, 0)),
        ],
        out_shape=[
            jax.ShapeDtypeStruct((1, 1), jnp.float32),
            jax.ShapeDtypeStruct((1, 1), jnp.float32),
        ],
    )(z_flat, zq, cnt)


def kernel(z_real, z_imag, W):
    z_flat = jnp.concatenate([z_real, z_imag], axis=-1)
    idx = _argmin_call(z_flat, W).reshape(N)
    zeros_k = jnp.zeros((K,), jnp.float32)
    ones_b = jnp.ones((BPW,), jnp.float32)
    zq, cnt = _gather_hist_call(W, idx, zeros_k, ones_b)
    loss, perp = _loss_call(z_flat, zq, cnt)
    z_q_c = lax.complex(zq[:, :DIM], zq[:, DIM:])
    return z_q_c, loss.reshape(()), perp.reshape(())


# speedup vs baseline: 1.3682x; 1.3682x over previous
import functools

import jax
import jax.numpy as jnp
from jax import lax
from jax.experimental import pallas as pl
from jax.experimental.pallas import tpu as pltpu
from jax.experimental.pallas import tpu_sc as plsc

N, DIM, K = 4096, 32, 8192
D2 = 2 * DIM
NT = 512
KC = 1024

NC = 2
NS = 16
NW = NC * NS
BPW = N // NW


NL = 128


def _argmin_body(z_ref, w_ref, idx_ref, msum_ref):
    z = z_ref[...]
    x2 = jnp.sum(z * z, axis=1, keepdims=True)
    zm2 = z * (-2.0)
    m_st = jnp.full((NT, NL), jnp.inf, dtype=jnp.float32)
    s_st = jnp.full((NT, NL), jnp.inf, dtype=jnp.float32)
    i_st = jnp.zeros((NT, NL), dtype=jnp.int32)
    lane = lax.broadcasted_iota(jnp.int32, (NT, NL), 1)
    for j in range(K // KC):
        wc = w_ref[pl.ds(j * KC, KC), :]
        w2 = jnp.sum(wc * wc, axis=1).reshape(1, KC)
        sp = lax.dot_general(zm2, wc, (((1,), (1,)), ((), ())),
                             preferred_element_type=jnp.float32)
        d2 = (x2 + w2) + sp
        d2c = jnp.maximum(d2, 0.0)
        for b in range(KC // NL):
            col = lax.slice(d2c, (0, b * NL), (NT, (b + 1) * NL))
            ci = lane + (j * KC + b * NL)
            upd = col < m_st
            s_st = jnp.minimum(s_st, jnp.maximum(col, m_st))
            i_st = jnp.where(upd, ci, i_st)
            m_st = jnp.minimum(col, m_st)
    m_row = jnp.min(m_st, axis=1, keepdims=True)
    elig = m_st == m_row
    idx_row = jnp.min(jnp.where(elig, i_st, K), axis=1, keepdims=True)
    eq_cnt = jnp.sum(jnp.where(elig, 1, 0), axis=1, keepdims=True)
    sec_lanes = jnp.min(jnp.where(elig, jnp.inf, m_st), axis=1, keepdims=True)
    sec_carry = jnp.min(s_st, axis=1, keepdims=True)
    sec_row = jnp.minimum(sec_carry,
                          jnp.where(eq_cnt >= 2, m_row, sec_lanes))
    mbits = lax.bitcast_convert_type(m_row, jnp.int32)
    sbits = lax.bitcast_convert_type(sec_row, jnp.int32)
    flag = (sbits <= mbits + 8) & (sbits >= 0)
    nflag = jnp.sum(jnp.where(flag, 1, 0))
    idx_ref[...] = idx_row
    msum_ref[pl.ds(pl.program_id(0), 1), :] = jnp.sum(m_row).reshape(1, 1)

    @pl.when(nflag > 0)
    def _exact():
        best_d = jnp.full((NT, 1), jnp.inf, dtype=jnp.float32)
        best_i = jnp.zeros((NT, 1), dtype=jnp.int32)
        for j in range(K // KC):
            wc = w_ref[pl.ds(j * KC, KC), :]
            w2 = jnp.sum(wc * wc, axis=1).reshape(1, KC)
            s = lax.dot_general(z, wc, (((1,), (1,)), ((), ())),
                                preferred_element_type=jnp.float32)
            d2 = (x2 + w2) - 2.0 * s
            d = jnp.sqrt(jnp.maximum(d2, 0.0))
            m = jnp.min(d, axis=1, keepdims=True)
            iota = lax.broadcasted_iota(jnp.int32, (NT, KC), 1) + j * KC
            cand = jnp.min(jnp.where(d == m, iota, K), axis=1, keepdims=True)
            updx = m < best_d
            best_d = jnp.where(updx, m, best_d)
            best_i = jnp.where(updx, cand, best_i)
        idx_ref[...] = best_i


def _argmin_call(z_flat, W):
    return pl.pallas_call(
        _argmin_body,
        grid=(N // NT,),
        in_specs=[
            pl.BlockSpec((NT, D2), lambda i: (i, 0)),
            pl.BlockSpec((K, D2), lambda i: (0, 0)),
        ],
        out_specs=[
            pl.BlockSpec((NT, 1), lambda i: (i, 0)),
            pl.BlockSpec((N // NT, 1), lambda i: (0, 0)),
        ],
        out_shape=[
            jax.ShapeDtypeStruct((N, 1), jnp.int32),
            jax.ShapeDtypeStruct((N // NT, 1), jnp.float32),
        ],
    )(z_flat, W)


def _gather_hist_body(w_hbm, idx_hbm, zeros_hbm, ones_hbm,
                      zq_out, cnt_out,
                      idx_v, rows_v, ones_v, cnt_sh, sem):
    c = lax.axis_index("c")
    s = lax.axis_index("s")
    wid = s * NC + c
    base = wid * BPW
    pltpu.sync_copy(idx_hbm.at[pl.ds(base, BPW)], idx_v)
    pltpu.async_copy(w_hbm.at[idx_v], rows_v, sem).wait()
    pltpu.sync_copy(rows_v, zq_out.at[pl.ds(base, BPW)])
    pltpu.sync_copy(ones_hbm, ones_v)

    @pl.when(s == 0)
    def _():
        pltpu.sync_copy(zeros_hbm, cnt_sh)

    plsc.subcore_barrier()
    pltpu.sync_copy(ones_v, cnt_sh.at[idx_v], add=True)
    plsc.subcore_barrier()

    @pl.when(s == 0)
    def _():
        pltpu.sync_copy(cnt_sh, cnt_out.at[c])


def _gather_hist_call(W, idx, zeros_k, ones_b):
    fn = functools.partial(
        pl.kernel,
        mesh=plsc.VectorSubcoreMesh(core_axis_name="c", subcore_axis_name="s"),
        out_type=[
            jax.ShapeDtypeStruct((N, D2), jnp.float32),
            jax.ShapeDtypeStruct((NC, K), jnp.float32),
        ],
        scratch_types=[
            pltpu.VMEM((BPW,), jnp.int32),
            pltpu.VMEM((BPW, D2), jnp.float32),
            pltpu.VMEM((BPW,), jnp.float32),
            pltpu.VMEM_SHARED((K,), jnp.float32),
            pltpu.SemaphoreType.DMA,
        ],
        compiler_params=pltpu.CompilerParams(use_tc_tiling_on_sc=False),
    )(_gather_hist_body)
    return fn(W, idx, zeros_k, ones_b)


def _loss_body(msum_ref, cnt_ref, loss_ref, perp_ref):
    tot = jnp.sum(msum_ref[...])
    loss_ref[...] = (jnp.float32(1.25 / (N * D2)) * tot).reshape(1, 1)
    p = jnp.sum(cnt_ref[...], axis=0, keepdims=True) * (1.0 / N)
    ent = jnp.sum(p * jnp.log(p + 1e-10))
    perp_ref[...] = jnp.exp(-ent).reshape(1, 1)


def _loss_call(msum, cnt):
    return pl.pallas_call(
        _loss_body,
        in_specs=[
            pl.BlockSpec((N // NT, 1), lambda: (0, 0)),
            pl.BlockSpec((NC, K), lambda: (0, 0)),
        ],
        out_specs=[
            pl.BlockSpec((1, 1), lambda: (0, 0)),
            pl.BlockSpec((1, 1), lambda: (0, 0)),
        ],
        out_shape=[
            jax.ShapeDtypeStruct((1, 1), jnp.float32),
            jax.ShapeDtypeStruct((1, 1), jnp.float32),
        ],
    )(msum, cnt)


def kernel(z_real, z_imag, W):
    z_flat = jnp.concatenate([z_real, z_imag], axis=-1)
    idx2d, msum = _argmin_call(z_flat, W)
    idx = idx2d.reshape(N)
    zeros_k = jnp.zeros((K,), jnp.float32)
    ones_b = jnp.ones((BPW,), jnp.float32)
    zq, cnt = _gather_hist_call(W, idx, zeros_k, ones_b)
    loss, perp = _loss_call(msum, cnt)
    z_q_c = lax.complex(zq[:, :DIM], zq[:, DIM:])
    return z_q_c, loss.reshape(()), perp.reshape(())
